# Initial kernel scaffold; baseline (speedup 1.0000x reference)
import jax
import jax.numpy as jnp
from jax.experimental import pallas as pl


def kernel(indices, graph_x, kernel, W0, W1, Wlin):
    raise NotImplementedError("write your pallas kernel here")


# speedup vs baseline: 47.8707x; 47.8707x over previous
import jax
import jax.numpy as jnp
from jax.experimental import pallas as pl
from jax.experimental.pallas import tpu as pltpu

_NODES = 26
_G = 8


def _gcn_step(idx_ref, *refs):
    gx = refs[:_G]
    kk = refs[_G:2 * _G]
    w0_ref, w1_ref, wl_ref, out_ref = refs[2 * _G:2 * _G + 4]
    w0 = w0_ref[...]
    w1 = w1_ref[...]
    wl = wl_ref[...]
    outs = []
    for g in range(_G):
        x = gx[g][0]
        K = kk[g][0]
        deg_r = jnp.sum(K, axis=1, keepdims=True)
        deg_c = jnp.sum(K, axis=0, keepdims=True)
        A = K * jax.lax.rsqrt(deg_r) * jax.lax.rsqrt(deg_c)
        h0 = jnp.dot(x, w0, preferred_element_type=jnp.float32)
        h1 = jnp.maximum(
            jnp.dot(A, h0, preferred_element_type=jnp.float32), 0.0)
        g1 = jnp.dot(h1, w1, preferred_element_type=jnp.float32)
        c = A[0:1, :]
        h2c = jnp.maximum(
            jnp.dot(c, g1, preferred_element_type=jnp.float32), 0.0)
        o = jnp.maximum(
            jnp.dot(h2c, wl, preferred_element_type=jnp.float32), 0.0)
        outs.append(o)
    out_ref[...] = jnp.concatenate(outs, axis=0)


def kernel(indices, graph_x, kernel, W0, W1, Wlin):
    B = indices.shape[0]
    n, nodes, in_dim = graph_x.shape
    h0 = W0.shape[1]
    h1 = W1.shape[1]
    out_dim = Wlin.shape[1]

    def gx_map(g):
        return lambda i, idx: (idx[i * _G + g], 0, 0)

    grid_spec = pltpu.PrefetchScalarGridSpec(
        num_scalar_prefetch=1,
        grid=(B // _G,),
        in_specs=(
            [pl.BlockSpec((1, nodes, in_dim), gx_map(g)) for g in range(_G)]
            + [pl.BlockSpec((1, nodes, nodes), gx_map(g)) for g in range(_G)]
            + [
                pl.BlockSpec((in_dim, h0), lambda i, idx: (0, 0)),
                pl.BlockSpec((h0, h1), lambda i, idx: (0, 0)),
                pl.BlockSpec((h1, out_dim), lambda i, idx: (0, 0)),
            ]
        ),
        out_specs=pl.BlockSpec((_G, out_dim), lambda i, idx: (i, 0)),
    )
    return pl.pallas_call(
        _gcn_step,
        grid_spec=grid_spec,
        out_shape=jax.ShapeDtypeStruct((B, out_dim), jnp.float32),
    )(indices, *([graph_x] * _G), *([kernel] * _G), W0, W1, Wlin)


# speedup vs baseline: 113.2887x; 2.3666x over previous
import functools

import jax
import jax.numpy as jnp
from jax import lax
from jax.experimental import pallas as pl
from jax.experimental.pallas import tpu as pltpu
from jax.experimental.pallas import tpu_sc as plsc

_NODES = 26
_KPAD = 64
_B = 1024
_NC, _NS = 2, 16
_NW = _NC * _NS
_BPW = _B // _NW
_XCHUNK = 16
_BT = 64
_RT = _BT * _NODES
_GRP = 16
_RG = _GRP * _NODES
_F32 = jnp.float32


def _sc_gather(x2d, k2d, indices):
    xd = x2d.shape[1]
    kd = k2d.shape[1]
    mesh = plsc.VectorSubcoreMesh(core_axis_name="c", subcore_axis_name="s")

    @functools.partial(
        pl.kernel,
        mesh=mesh,
        out_type=(
            jax.ShapeDtypeStruct((_B, xd), _F32),
            jax.ShapeDtypeStruct((_B, kd), _F32),
        ),
        scratch_types=[
            pltpu.VMEM((_XCHUNK,), jnp.int32),
            pltpu.VMEM((_XCHUNK, xd), _F32),
            pltpu.VMEM((_XCHUNK, kd), _F32),
            pltpu.SemaphoreType.DMA,
        ],
    )
    def gather_kernel(x_hbm, k_hbm, idx_hbm, xg_hbm, kg_hbm,
                      idx_x, xrows, krows, sem):
        wid = lax.axis_index("s") * _NC + lax.axis_index("c")
        base = wid * _BPW
        for c in range(_BPW // _XCHUNK):
            off = base + c * _XCHUNK
            pltpu.sync_copy(idx_hbm.at[pl.ds(off, _XCHUNK)], idx_x)
            pltpu.async_copy(x_hbm.at[idx_x], xrows, sem).wait()
            pltpu.sync_copy(xrows, xg_hbm.at[pl.ds(off, _XCHUNK)])
            pltpu.async_copy(k_hbm.at[idx_x], krows, sem).wait()
            pltpu.sync_copy(krows, kg_hbm.at[pl.ds(off, _XCHUNK)])

    return gather_kernel(x2d, k2d, indices)


def _tc_body(xb_ref, kb_ref, w0_ref, w1_ref, wl_ref, out_ref,
             s_ref, p_ref, m_ref):
    @pl.when(pl.program_id(0) == 0)
    def _init_masks():
        rowv = lax.broadcasted_iota(jnp.int32, (_BT, _RT), 0) * _NODES
        colv = lax.broadcasted_iota(jnp.int32, (_BT, _RT), 1)
        s_ref[...] = ((colv >= rowv) & (colv < rowv + _NODES)).astype(_F32)
        ir = lax.broadcasted_iota(jnp.int32, (_NODES, _NODES), 0)
        ic = lax.broadcasted_iota(jnp.int32, (_NODES, _NODES), 1)
        eye = (ir == ic).astype(_F32)
        for gj in range(_GRP):
            p_ref[:, gj * _NODES:(gj + 1) * _NODES] = eye
        rv = lax.broadcasted_iota(jnp.int32, (_GRP, _RG), 0) * _NODES
        cv = lax.broadcasted_iota(jnp.int32, (_GRP, _RG), 1)
        s16 = ((cv >= rv) & (cv < rv + _NODES)).astype(_F32)
        m_ref[...] = lax.dot_general(
            s16, s16, (((0,), (0,)), ((), ())),
            preferred_element_type=_F32)

    xb = xb_ref[...]
    kb = kb_ref[...][:, :_NODES]
    w0 = w0_ref[...]
    w1 = w1_ref[...]
    wl = wl_ref[...]
    s_mask = s_ref[...]
    p_tile = p_ref[...]
    m_mask = m_ref[...]

    dinv_r = lax.rsqrt(jnp.sum(kb, axis=1, keepdims=True))
    colsum = jnp.dot(s_mask, kb, preferred_element_type=_F32)
    dinv_b = lax.rsqrt(colsum)
    dinv_c = lax.dot_general(
        s_mask, dinv_b, (((0,), (0,)), ((), ())),
        preferred_element_type=_F32)
    a_rows = kb * dinv_r * dinv_c

    h0 = jnp.dot(xb, w0, preferred_element_type=_F32)

    h1_parts = []
    for g in range(_BT // _GRP):
        rows = slice(g * _RG, (g + 1) * _RG)
        bd = jnp.dot(a_rows[rows], p_tile,
                     preferred_element_type=_F32) * m_mask
        agg = lax.dot_general(
            bd, h0[rows], (((0,), (0,)), ((), ())),
            preferred_element_type=_F32)
        h1_parts.append(jnp.maximum(agg, 0.0))
    h1 = jnp.concatenate(h1_parts, axis=0)

    g1 = jnp.dot(h1, w1, preferred_element_type=_F32)
    wg1 = g1 * a_rows[:, 0:1]
    centers = jnp.maximum(
        jnp.dot(s_mask, wg1, preferred_element_type=_F32), 0.0)
    out_ref[...] = jnp.maximum(
        jnp.dot(centers, wl, preferred_element_type=_F32), 0.0)


def _tc_compute(xg2, kg2, W0, W1, Wlin):
    in_dim = xg2.shape[1]
    h0 = W0.shape[1]
    h1 = W1.shape[1]
    od = Wlin.shape[1]
    return pl.pallas_call(
        _tc_body,
        grid=(_B // _BT,),
        in_specs=[
            pl.BlockSpec((_RT, in_dim), lambda i: (i, 0)),
            pl.BlockSpec((_RT, _KPAD), lambda i: (i, 0)),
            pl.BlockSpec((in_dim, h0), lambda i: (0, 0)),
            pl.BlockSpec((h0, h1), lambda i: (0, 0)),
            pl.BlockSpec((h1, od), lambda i: (0, 0)),
        ],
        out_specs=pl.BlockSpec((_BT, od), lambda i: (i, 0)),
        out_shape=jax.ShapeDtypeStruct((_B, od), _F32),
        scratch_shapes=[
            pltpu.VMEM((_BT, _RT), _F32),
            pltpu.VMEM((_NODES, _RG), _F32),
            pltpu.VMEM((_RG, _RG), _F32),
        ],
    )(xg2, kg2, W0, W1, Wlin)


def kernel(indices, graph_x, kernel, W0, W1, Wlin):
    n, nodes, in_dim = graph_x.shape
    x2d = graph_x.reshape(n, nodes * in_dim)
    kpad = jnp.pad(kernel, ((0, 0), (0, 0), (0, _KPAD - nodes)))
    k2d = kpad.reshape(n, nodes * _KPAD)
    xg, kg = _sc_gather(x2d, k2d, indices)
    xg2 = xg.reshape(_B * nodes, in_dim)
    kg2 = kg.reshape(_B * nodes, _KPAD)
    return _tc_compute(xg2, kg2, W0, W1, Wlin)
